# baseline (speedup 1.0000x reference)
import functools

import jax
import jax.numpy as jnp
from jax import lax
from jax.experimental import pallas as pl
from jax.experimental.pallas import tpu as pltpu
from jax.experimental.pallas import tpu_sc as plsc

N_NODES = 10000
N_GRAPHS = 64
N_EDGES = 320000

E_PAD = 327680
IDX_ROWS = E_PAD // 128
CAP_ROWS = 1024
CAP_E = CAP_ROWS * 128
SEG = 3392
NSEG = 3
ACC_ROWS = 3456
DEG_ROWS = 10240
POOL_BLK = 2000

_MESH = plsc.VectorSubcoreMesh(core_axis_name="c", subcore_axis_name="s")


def _deg_sc(dst2d):
    nrt = IDX_ROWS // 32
    zr = DEG_ROWS // 16

    @functools.partial(
        pl.kernel,
        out_type=[
            jax.ShapeDtypeStruct((DEG_ROWS,), jnp.float32),
            jax.ShapeDtypeStruct((DEG_ROWS,), jnp.float32),
        ],
        mesh=_MESH,
        scratch_types=[
            pltpu.VMEM((nrt, 128), jnp.int32),
            pltpu.VMEM((128,), jnp.float32),
            pltpu.VMEM((zr,), jnp.float32),
            pltpu.VMEM_SHARED((DEG_ROWS,), jnp.float32),
        ],
    )
    def deg_kernel(dst_hbm, out0, out1, didx, ones, zbuf, acc):
        c = lax.axis_index("c")
        s = lax.axis_index("s")

        def fill(i, carry):
            zbuf[pl.ds(i * 16, 16)] = jnp.zeros((16,), jnp.float32)
            return carry

        lax.fori_loop(0, zr // 16, fill, 0)
        for j in range(8):
            ones[pl.ds(j * 16, 16)] = jnp.ones((16,), jnp.float32)

        pltpu.sync_copy(zbuf, acc.at[pl.ds(s * zr, zr)])
        wbase = (c * 16 + s) * nrt
        pltpu.sync_copy(dst_hbm.at[pl.ds(wbase, nrt)], didx)
        plsc.subcore_barrier()

        def body(j, carry):
            pltpu.sync_copy(ones, acc.at[didx.at[j]], add=True)
            return carry

        lax.fori_loop(0, nrt, body, 0)
        plsc.subcore_barrier()

        @pl.when(c == 0)
        def _w0():
            pltpu.sync_copy(acc.at[pl.ds(s * zr, zr)], out0.at[pl.ds(s * zr, zr)])

        @pl.when(c == 1)
        def _w1():
            pltpu.sync_copy(acc.at[pl.ds(s * zr, zr)], out1.at[pl.ds(s * zr, zr)])

    return deg_kernel(dst2d)


def _agg_body(tab_for_core, src_hbm, dst_hbm, outs,
              sidx, didx, rows, zbuf, acc, sem0, sem1, *, nrt, edge_split):
    c = lax.axis_index("c")
    s = lax.axis_index("s")
    f = zbuf.shape[1]
    zr = ACC_ROWS // 16

    def fill(i, carry):
        for j in range(f // 16):
            zbuf[i, pl.ds(j * 16, 16)] = jnp.zeros((16,), jnp.float32)
        return carry

    lax.fori_loop(0, zr, fill, 0)
    base = s * zr
    pltpu.sync_copy(zbuf, acc.at[pl.ds(base, zr)])

    if edge_split:
        rbase = (c * 16 + s) * nrt
    else:
        rbase = s * nrt
    pltpu.sync_copy(src_hbm.at[pl.ds(rbase, nrt)], sidx)
    pltpu.sync_copy(dst_hbm.at[pl.ds(rbase, nrt)], didx)
    plsc.subcore_barrier()

    def edge_loop(tab):
        pltpu.async_copy(tab.at[sidx.at[0]], rows.at[0], sem0)

        def body(i, carry):
            j0 = 2 * i
            pltpu.make_async_copy(tab.at[sidx.at[j0]], rows.at[0], sem0).wait()
            pltpu.async_copy(tab.at[sidx.at[j0 + 1]], rows.at[1], sem1)
            pltpu.sync_copy(rows.at[0], acc.at[didx.at[j0]], add=True)
            pltpu.make_async_copy(
                tab.at[sidx.at[j0 + 1]], rows.at[1], sem1).wait()

            @pl.when(j0 + 2 < nrt)
            def _prefetch():
                pltpu.async_copy(tab.at[sidx.at[j0 + 2]], rows.at[0], sem0)

            pltpu.sync_copy(rows.at[1], acc.at[didx.at[j0 + 1]], add=True)
            return carry

        lax.fori_loop(0, nrt // 2, body, 0)

    if edge_split:
        edge_loop(tab_for_core[0])
    else:
        @pl.when(c == 0)
        def _e0():
            edge_loop(tab_for_core[0])

        @pl.when(c == 1)
        def _e1():
            edge_loop(tab_for_core[1])

    plsc.subcore_barrier()

    @pl.when(c == 0)
    def _w0():
        pltpu.sync_copy(acc.at[pl.ds(base, zr)], outs[0].at[pl.ds(base, zr)])

    @pl.when(c == 1)
    def _w1():
        pltpu.sync_copy(acc.at[pl.ds(base, zr)], outs[1].at[pl.ds(base, zr)])


def _agg_scratch(f, nrt):
    return [
        pltpu.VMEM((nrt, 128), jnp.int32),
        pltpu.VMEM((nrt, 128), jnp.int32),
        pltpu.VMEM((2, 128, f), jnp.float32),
        pltpu.VMEM((ACC_ROWS // 16, f), jnp.float32),
        pltpu.VMEM_SHARED((ACC_ROWS, f), jnp.float32),
        pltpu.SemaphoreType.DMA,
        pltpu.SemaphoreType.DMA,
    ]


def _agg_sc(g_lo, g_hi, src2d, dsts2d):
    f = g_lo.shape[1]
    nrt = src2d.shape[0] // 16

    @functools.partial(
        pl.kernel,
        out_type=[jax.ShapeDtypeStruct((ACC_ROWS, f), jnp.float32)] * 2,
        mesh=_MESH,
        scratch_types=_agg_scratch(f, nrt),
    )
    def agg_kernel(glo_hbm, ghi_hbm, src_hbm, dst_hbm, out_lo, out_hi,
                   sidx, didx, rows, zbuf, acc, sem0, sem1):
        _agg_body((glo_hbm, ghi_hbm), src_hbm, dst_hbm, (out_lo, out_hi),
                  sidx, didx, rows, zbuf, acc, sem0, sem1,
                  nrt=nrt, edge_split=False)

    return agg_kernel(g_lo, g_hi, src2d, dsts2d)


def _agg_sc_esplit(g, src2d, dsts2d):
    f = g.shape[1]
    nrt = src2d.shape[0] // 32

    @functools.partial(
        pl.kernel,
        out_type=[jax.ShapeDtypeStruct((ACC_ROWS, f), jnp.float32)] * 2,
        mesh=_MESH,
        scratch_types=_agg_scratch(f, nrt),
    )
    def agg_kernel(g_hbm, src_hbm, dst_hbm, out_a, out_b,
                   sidx, didx, rows, zbuf, acc, sem0, sem1):
        _agg_body((g_hbm,), src_hbm, dst_hbm, (out_a, out_b),
                  sidx, didx, rows, zbuf, acc, sem0, sem1,
                  nrt=nrt, edge_split=True)

    return agg_kernel(g, src2d, dsts2d)


def _dinv_body(x_ref, d0_ref, d1_ref, dinv_ref, g_ref):
    dinv = lax.rsqrt(1.0 + d0_ref[...] + d1_ref[...])
    dinv_ref[...] = dinv
    g_ref[...] = dinv * x_ref[...]


def _dinv_tc(x, d0, d1):
    n, fin = x.shape
    return pl.pallas_call(
        _dinv_body,
        grid=(NSEG,),
        in_specs=[
            pl.BlockSpec((SEG, fin), lambda i: (i, 0)),
            pl.BlockSpec((SEG, 1), lambda i: (i, 0)),
            pl.BlockSpec((SEG, 1), lambda i: (i, 0)),
        ],
        out_specs=[
            pl.BlockSpec((SEG, 1), lambda i: (i, 0)),
            pl.BlockSpec((SEG, fin), lambda i: (i, 0)),
        ],
        out_shape=[
            jax.ShapeDtypeStruct((n, 1), jnp.float32),
            jax.ShapeDtypeStruct((n, fin), jnp.float32),
        ],
    )(x, d0, d1)


def _seg_spec(f):
    return pl.BlockSpec((SEG, f), lambda i: (0, 0))


def _pick3(i, r0, r1, r2):
    return jnp.where(i == 0, r0[...], jnp.where(i == 1, r1[...], r2[...]))


def _layer1_body(s0a_ref, s0b_ref, s1a_ref, s1b_ref, s2a_ref, s2b_ref,
                 g_ref, dinv_ref, w_ref, b_ref, glo_ref, ghi_ref):
    i = pl.program_id(0)
    sa = _pick3(i, s0a_ref, s1a_ref, s2a_ref)
    sb = _pick3(i, s0b_ref, s1b_ref, s2b_ref)
    dinv = dinv_ref[...]
    t = dinv * (sa + sb + g_ref[...])
    h = jnp.dot(t, w_ref[...], preferred_element_type=jnp.float32)
    h = jnp.maximum(h + b_ref[...], 0.0)
    g = dinv * h
    fo = h.shape[1] // 2
    glo_ref[...] = g[:, :fo]
    ghi_ref[...] = g[:, fo:]


def _layer1_tc(s_parts, g, dinv, W, b):
    n, fin = g.shape
    fout = W.shape[1]
    fo = fout // 2
    flat = [r for ab in s_parts for r in ab]
    return pl.pallas_call(
        _layer1_body,
        grid=(NSEG,),
        in_specs=[_seg_spec(fin)] * 6 + [
            pl.BlockSpec((SEG, fin), lambda i: (i, 0)),
            pl.BlockSpec((SEG, 1), lambda i: (i, 0)),
            pl.BlockSpec((fin, fout), lambda i: (0, 0)),
            pl.BlockSpec((1, fout), lambda i: (0, 0)),
        ],
        out_specs=[
            pl.BlockSpec((SEG, fo), lambda i: (i, 0)),
            pl.BlockSpec((SEG, fo), lambda i: (i, 0)),
        ],
        out_shape=[
            jax.ShapeDtypeStruct((n, fo), jnp.float32),
            jax.ShapeDtypeStruct((n, fo), jnp.float32),
        ],
    )(*flat, g, dinv, W, b)


def _layer_body(s0l_ref, s0h_ref, s1l_ref, s1h_ref, s2l_ref, s2h_ref,
                glo_ref, ghi_ref, dinv_ref, w_ref, b_ref, *out_refs,
                act, last):
    i = pl.program_id(0)
    s_lo = _pick3(i, s0l_ref, s1l_ref, s2l_ref)
    s_hi = _pick3(i, s0h_ref, s1h_ref, s2h_ref)
    dinv = dinv_ref[...]
    hf = s_lo.shape[1]
    t_lo = dinv * (s_lo + glo_ref[...])
    t_hi = dinv * (s_hi + ghi_ref[...])
    h = jnp.dot(t_lo, w_ref[:hf, :], preferred_element_type=jnp.float32)
    h += jnp.dot(t_hi, w_ref[hf:, :], preferred_element_type=jnp.float32)
    h = h + b_ref[...]
    if act:
        h = jnp.maximum(h, 0.0)
    if last:
        out_refs[0][...] = h
    else:
        fo = h.shape[1] // 2
        g = dinv * h
        out_refs[0][...] = g[:, :fo]
        out_refs[1][...] = g[:, fo:]


def _layer_tc(s_parts, g_lo, g_hi, dinv, W, b, act, last):
    n = g_lo.shape[0]
    hf = g_lo.shape[1]
    fout = W.shape[1]
    fo = fout // 2
    if last:
        out_specs = [pl.BlockSpec((SEG, fout), lambda i: (i, 0))]
        out_shape = [jax.ShapeDtypeStruct((n, fout), jnp.float32)]
    else:
        out_specs = [
            pl.BlockSpec((SEG, fo), lambda i: (i, 0)),
            pl.BlockSpec((SEG, fo), lambda i: (i, 0)),
        ]
        out_shape = [
            jax.ShapeDtypeStruct((n, fo), jnp.float32),
            jax.ShapeDtypeStruct((n, fo), jnp.float32),
        ]
    flat = [r for lh in s_parts for r in lh]
    return pl.pallas_call(
        functools.partial(_layer_body, act=act, last=last),
        grid=(NSEG,),
        in_specs=[_seg_spec(hf)] * 6 + [
            pl.BlockSpec((SEG, hf), lambda i: (i, 0)),
            pl.BlockSpec((SEG, hf), lambda i: (i, 0)),
            pl.BlockSpec((SEG, 1), lambda i: (i, 0)),
            pl.BlockSpec((2 * hf, fout), lambda i: (0, 0)),
            pl.BlockSpec((1, fout), lambda i: (0, 0)),
        ],
        out_specs=out_specs,
        out_shape=out_shape,
    )(*flat, g_lo, g_hi, dinv, W, b)


def _pool_body(h_ref, batch_ref, wl_ref, bl_ref, out_ref, acc_ref, cnt_ref):
    i = pl.program_id(0)

    @pl.when(i == 0)
    def _init():
        acc_ref[...] = jnp.zeros_like(acc_ref)
        cnt_ref[...] = jnp.zeros_like(cnt_ref)

    b = batch_ref[0, 0, :]
    gids = lax.broadcasted_iota(jnp.int32, (N_GRAPHS, POOL_BLK), 0)
    onehot = (gids == b[None, :]).astype(jnp.float32)
    acc_ref[...] += jnp.dot(onehot, h_ref[...], preferred_element_type=jnp.float32)
    cnt_ref[...] += jnp.sum(onehot, axis=1, keepdims=True)

    @pl.when(i == pl.num_programs(0) - 1)
    def _fin():
        pooled = acc_ref[...] / jnp.maximum(cnt_ref[...], 1.0)
        out_ref[...] = (
            jnp.dot(pooled, wl_ref[...], preferred_element_type=jnp.float32)
            + bl_ref[...]
        )


def _pool_classify(h, batch, Wl, bl):
    n, f = h.shape
    ncls = Wl.shape[1]
    batch3d = batch.astype(jnp.int32).reshape(n // POOL_BLK, 1, POOL_BLK)
    return pl.pallas_call(
        _pool_body,
        grid=(n // POOL_BLK,),
        in_specs=[
            pl.BlockSpec((POOL_BLK, f), lambda i: (i, 0)),
            pl.BlockSpec((1, 1, POOL_BLK), lambda i: (i, 0, 0)),
            pl.BlockSpec((f, ncls), lambda i: (0, 0)),
            pl.BlockSpec((1, ncls), lambda i: (0, 0)),
        ],
        out_specs=pl.BlockSpec((N_GRAPHS, ncls), lambda i: (0, 0)),
        out_shape=jax.ShapeDtypeStruct((N_GRAPHS, ncls), jnp.float32),
        scratch_shapes=[
            pltpu.VMEM((N_GRAPHS, f), jnp.float32),
            pltpu.VMEM((N_GRAPHS, 1), jnp.float32),
        ],
    )(h, batch3d, Wl, bl.reshape(1, ncls))


def kernel(x, edge_index, batch, W1, b1, W2, b2, W3, b3, W4, b4, Wl, bl):
    src = edge_index[0].astype(jnp.int32)
    dst = edge_index[1].astype(jnp.int32)
    npad = E_PAD - N_EDGES
    pad = jnp.arange(npad, dtype=jnp.int32) % 16
    src_p = jnp.concatenate([src, pad])
    dst_p = jnp.concatenate([dst, N_NODES + pad])
    key = (dst_p >= SEG).astype(jnp.int32) + (dst_p >= 2 * SEG).astype(jnp.int32)
    _, src_p, dst_p = lax.sort((key, src_p, dst_p), num_keys=1)
    src2d = src_p.reshape(IDX_ROWS, 128)
    dst2d = dst_p.reshape(IDX_ROWS, 128)
    dump = SEG + (jnp.arange(E_PAD, dtype=jnp.int32) % (ACC_ROWS - SEG))
    dsts = []
    for k in range(NSEG):
        lo = k * SEG
        hi = min(lo + SEG, N_NODES)
        in_seg = (dst_p >= lo) & (dst_p < hi)
        dsts.append(jnp.where(in_seg, dst_p - lo, dump).reshape(IDX_ROWS, 128))

    n0 = jnp.sum((key == 0).astype(jnp.int32))
    n1 = jnp.sum((key == 1).astype(jnp.int32))
    n2 = E_PAD - n0 - n1
    starts = [jnp.int32(0), n0, n0 + n1]
    rks = [jnp.clip(st // 128, 0, IDX_ROWS - CAP_ROWS) for st in starts]
    src_w = [lax.dynamic_slice(src2d, (rk, 0), (CAP_ROWS, 128)) for rk in rks]
    dst_w = [lax.dynamic_slice(dsts[k], (rks[k], 0), (CAP_ROWS, 128))
             for k in range(NSEG)]
    fits = (n0 <= CAP_E - 128) & (n1 <= CAP_E - 128) & (n2 <= CAP_E - 128)

    deg0, deg1 = _deg_sc(dst2d)
    d0 = deg0[:N_NODES].reshape(N_NODES, 1)
    d1 = deg1[:N_NODES].reshape(N_NODES, 1)

    dinv, g0 = _dinv_tc(x, d0, d1)

    def _chain(agg1, agg):
        s_parts = agg1(g0)
        g_lo, g_hi = _layer1_tc(s_parts, g0, dinv, W1, b1.reshape(1, -1))
        layers = ((W2, b2, True), (W3, b3, True), (W4, b4, False))
        h = None
        for li, (W, b, act) in enumerate(layers):
            s_parts = agg(g_lo, g_hi)
            last = li == len(layers) - 1
            outs = _layer_tc(s_parts, g_lo, g_hi, dinv, W, b.reshape(1, -1),
                             act=act, last=last)
            if last:
                h = outs[0]
            else:
                g_lo, g_hi = outs
        return h

    def _fast():
        return _chain(
            lambda g: [_agg_sc_esplit(g, src_w[k], dst_w[k])
                       for k in range(NSEG)],
            lambda gl, gh: [_agg_sc(gl, gh, src_w[k], dst_w[k])
                            for k in range(NSEG)],
        )

    def _slow():
        return _chain(
            lambda g: [_agg_sc_esplit(g, src2d, dsts[k])
                       for k in range(NSEG)],
            lambda gl, gh: [_agg_sc(gl, gh, src2d, dsts[k])
                            for k in range(NSEG)],
        )

    h = lax.cond(fits, _fast, _slow)
    return _pool_classify(h, batch, Wl, bl)


# speedup vs baseline: 1.0896x; 1.0896x over previous
import functools

import jax
import jax.numpy as jnp
from jax import lax
from jax.experimental import pallas as pl
from jax.experimental.pallas import tpu as pltpu
from jax.experimental.pallas import tpu_sc as plsc

N_NODES = 10000
N_GRAPHS = 64
N_EDGES = 320000

E_PAD = 327680
IDX_ROWS = E_PAD // 128
CAP_ROWS = 1024
CAP_E = CAP_ROWS * 128
SEG = 3392
NSEG = 3
ACC_ROWS = 3456
DEG_ROWS = 10240
POOL_BLK = 2000

_MESH = plsc.VectorSubcoreMesh(core_axis_name="c", subcore_axis_name="s")


def _deg_sc(dst2d):
    nrt = IDX_ROWS // 32
    zr = DEG_ROWS // 16

    @functools.partial(
        pl.kernel,
        out_type=[
            jax.ShapeDtypeStruct((DEG_ROWS,), jnp.float32),
            jax.ShapeDtypeStruct((DEG_ROWS,), jnp.float32),
        ],
        mesh=_MESH,
        scratch_types=[
            pltpu.VMEM((nrt, 128), jnp.int32),
            pltpu.VMEM((128,), jnp.float32),
            pltpu.VMEM((zr,), jnp.float32),
            pltpu.VMEM_SHARED((DEG_ROWS,), jnp.float32),
        ],
    )
    def deg_kernel(dst_hbm, out0, out1, didx, ones, zbuf, acc):
        c = lax.axis_index("c")
        s = lax.axis_index("s")

        def fill(i, carry):
            zbuf[pl.ds(i * 16, 16)] = jnp.zeros((16,), jnp.float32)
            return carry

        lax.fori_loop(0, zr // 16, fill, 0)
        for j in range(8):
            ones[pl.ds(j * 16, 16)] = jnp.ones((16,), jnp.float32)

        pltpu.sync_copy(zbuf, acc.at[pl.ds(s * zr, zr)])
        wbase = (c * 16 + s) * nrt
        pltpu.sync_copy(dst_hbm.at[pl.ds(wbase, nrt)], didx)
        plsc.subcore_barrier()

        def body(j, carry):
            pltpu.sync_copy(ones, acc.at[didx.at[j]], add=True)
            return carry

        lax.fori_loop(0, nrt, body, 0)
        plsc.subcore_barrier()

        @pl.when(c == 0)
        def _w0():
            pltpu.sync_copy(acc.at[pl.ds(s * zr, zr)], out0.at[pl.ds(s * zr, zr)])

        @pl.when(c == 1)
        def _w1():
            pltpu.sync_copy(acc.at[pl.ds(s * zr, zr)], out1.at[pl.ds(s * zr, zr)])

    return deg_kernel(dst2d)


def _agg_body(tab_for_core, src_hbm, dst_hbm, outs,
              sidx, didx, rows, zbuf, acc, sem0, sem1, *, nrt, edge_split):
    c = lax.axis_index("c")
    s = lax.axis_index("s")
    f = zbuf.shape[1]
    zr = ACC_ROWS // 16

    def fill(i, carry):
        for j in range(f // 16):
            zbuf[i, pl.ds(j * 16, 16)] = jnp.zeros((16,), jnp.float32)
        return carry

    lax.fori_loop(0, zr, fill, 0)
    base = s * zr
    pltpu.sync_copy(zbuf, acc.at[pl.ds(base, zr)])

    if edge_split:
        rbase = (c * 16 + s) * nrt
    else:
        rbase = s * nrt
    pltpu.sync_copy(src_hbm.at[pl.ds(rbase, nrt)], sidx)
    pltpu.sync_copy(dst_hbm.at[pl.ds(rbase, nrt)], didx)
    plsc.subcore_barrier()

    def edge_loop(tab):
        pltpu.async_copy(tab.at[sidx.at[0]], rows.at[0], sem0)

        def body(i, carry):
            j0 = 2 * i
            pltpu.make_async_copy(tab.at[sidx.at[j0]], rows.at[0], sem0).wait()
            pltpu.async_copy(tab.at[sidx.at[j0 + 1]], rows.at[1], sem1)
            pltpu.sync_copy(rows.at[0], acc.at[didx.at[j0]], add=True)
            pltpu.make_async_copy(
                tab.at[sidx.at[j0 + 1]], rows.at[1], sem1).wait()

            @pl.when(j0 + 2 < nrt)
            def _prefetch():
                pltpu.async_copy(tab.at[sidx.at[j0 + 2]], rows.at[0], sem0)

            pltpu.sync_copy(rows.at[1], acc.at[didx.at[j0 + 1]], add=True)
            return carry

        lax.fori_loop(0, nrt // 2, body, 0)

    if edge_split:
        edge_loop(tab_for_core[0])
    else:
        @pl.when(c == 0)
        def _e0():
            edge_loop(tab_for_core[0])

        @pl.when(c == 1)
        def _e1():
            edge_loop(tab_for_core[1])

    plsc.subcore_barrier()

    @pl.when(c == 0)
    def _w0():
        pltpu.sync_copy(acc.at[pl.ds(base, zr)], outs[0].at[pl.ds(base, zr)])

    @pl.when(c == 1)
    def _w1():
        pltpu.sync_copy(acc.at[pl.ds(base, zr)], outs[1].at[pl.ds(base, zr)])


def _agg_scratch(f, nrt):
    return [
        pltpu.VMEM((nrt, 128), jnp.int32),
        pltpu.VMEM((nrt, 128), jnp.int32),
        pltpu.VMEM((2, 128, f), jnp.float32),
        pltpu.VMEM((ACC_ROWS // 16, f), jnp.float32),
        pltpu.VMEM_SHARED((ACC_ROWS, f), jnp.float32),
        pltpu.SemaphoreType.DMA,
        pltpu.SemaphoreType.DMA,
    ]


def _agg_sc(g_lo, g_hi, src2d, dsts2d):
    f = g_lo.shape[1]
    nrt = src2d.shape[0] // 16

    @functools.partial(
        pl.kernel,
        out_type=[jax.ShapeDtypeStruct((ACC_ROWS, f), jnp.float32)] * 2,
        mesh=_MESH,
        scratch_types=_agg_scratch(f, nrt),
    )
    def agg_kernel(glo_hbm, ghi_hbm, src_hbm, dst_hbm, out_lo, out_hi,
                   sidx, didx, rows, zbuf, acc, sem0, sem1):
        _agg_body((glo_hbm, ghi_hbm), src_hbm, dst_hbm, (out_lo, out_hi),
                  sidx, didx, rows, zbuf, acc, sem0, sem1,
                  nrt=nrt, edge_split=False)

    return agg_kernel(g_lo, g_hi, src2d, dsts2d)


def _agg_sc_esplit(g, src2d, dsts2d):
    f = g.shape[1]
    nrt = src2d.shape[0] // 32

    @functools.partial(
        pl.kernel,
        out_type=[jax.ShapeDtypeStruct((ACC_ROWS, f), jnp.float32)] * 2,
        mesh=_MESH,
        scratch_types=_agg_scratch(f, nrt),
    )
    def agg_kernel(g_hbm, src_hbm, dst_hbm, out_a, out_b,
                   sidx, didx, rows, zbuf, acc, sem0, sem1):
        _agg_body((g_hbm,), src_hbm, dst_hbm, (out_a, out_b),
                  sidx, didx, rows, zbuf, acc, sem0, sem1,
                  nrt=nrt, edge_split=True)

    return agg_kernel(g, src2d, dsts2d)


def _dinv_body(x_ref, d0_ref, d1_ref, dinv_ref, g_ref):
    dinv = lax.rsqrt(1.0 + d0_ref[...] + d1_ref[...])
    dinv_ref[...] = dinv
    g_ref[...] = dinv * x_ref[...]


def _dinv_tc(x, d0, d1):
    n, fin = x.shape
    return pl.pallas_call(
        _dinv_body,
        grid=(NSEG,),
        in_specs=[
            pl.BlockSpec((SEG, fin), lambda i: (i, 0)),
            pl.BlockSpec((SEG, 1), lambda i: (i, 0)),
            pl.BlockSpec((SEG, 1), lambda i: (i, 0)),
        ],
        out_specs=[
            pl.BlockSpec((SEG, 1), lambda i: (i, 0)),
            pl.BlockSpec((SEG, fin), lambda i: (i, 0)),
        ],
        out_shape=[
            jax.ShapeDtypeStruct((n, 1), jnp.float32),
            jax.ShapeDtypeStruct((n, fin), jnp.float32),
        ],
    )(x, d0, d1)


def _seg_spec(f):
    return pl.BlockSpec((SEG, f), lambda i: (0, 0))


def _pick3(i, r0, r1, r2):
    return jnp.where(i == 0, r0[...], jnp.where(i == 1, r1[...], r2[...]))


def _layer1_body(s0a_ref, s0b_ref, s1a_ref, s1b_ref, s2a_ref, s2b_ref,
                 g_ref, dinv_ref, w_ref, b_ref, glo_ref, ghi_ref):
    i = pl.program_id(0)
    sa = _pick3(i, s0a_ref, s1a_ref, s2a_ref)
    sb = _pick3(i, s0b_ref, s1b_ref, s2b_ref)
    dinv = dinv_ref[...]
    t = dinv * (sa + sb + g_ref[...])
    h = jnp.dot(t, w_ref[...], preferred_element_type=jnp.float32)
    h = jnp.maximum(h + b_ref[...], 0.0)
    g = dinv * h
    fo = h.shape[1] // 2
    glo_ref[...] = g[:, :fo]
    ghi_ref[...] = g[:, fo:]


def _layer1_tc(s_parts, g, dinv, W, b):
    n, fin = g.shape
    fout = W.shape[1]
    fo = fout // 2
    flat = [r for ab in s_parts for r in ab]
    return pl.pallas_call(
        _layer1_body,
        grid=(NSEG,),
        in_specs=[_seg_spec(fin)] * 6 + [
            pl.BlockSpec((SEG, fin), lambda i: (i, 0)),
            pl.BlockSpec((SEG, 1), lambda i: (i, 0)),
            pl.BlockSpec((fin, fout), lambda i: (0, 0)),
            pl.BlockSpec((1, fout), lambda i: (0, 0)),
        ],
        out_specs=[
            pl.BlockSpec((SEG, fo), lambda i: (i, 0)),
            pl.BlockSpec((SEG, fo), lambda i: (i, 0)),
        ],
        out_shape=[
            jax.ShapeDtypeStruct((n, fo), jnp.float32),
            jax.ShapeDtypeStruct((n, fo), jnp.float32),
        ],
    )(*flat, g, dinv, W, b)


def _layer_body(s0l_ref, s0h_ref, s1l_ref, s1h_ref, s2l_ref, s2h_ref,
                glo_ref, ghi_ref, dinv_ref, w_ref, b_ref, *out_refs,
                act, last):
    i = pl.program_id(0)
    s_lo = _pick3(i, s0l_ref, s1l_ref, s2l_ref)
    s_hi = _pick3(i, s0h_ref, s1h_ref, s2h_ref)
    dinv = dinv_ref[...]
    hf = s_lo.shape[1]
    t_lo = dinv * (s_lo + glo_ref[...])
    t_hi = dinv * (s_hi + ghi_ref[...])
    h = jnp.dot(t_lo, w_ref[:hf, :], preferred_element_type=jnp.float32)
    h += jnp.dot(t_hi, w_ref[hf:, :], preferred_element_type=jnp.float32)
    h = h + b_ref[...]
    if act:
        h = jnp.maximum(h, 0.0)
    if last:
        out_refs[0][...] = h
    else:
        fo = h.shape[1] // 2
        g = dinv * h
        out_refs[0][...] = g[:, :fo]
        out_refs[1][...] = g[:, fo:]


def _layer_tc(s_parts, g_lo, g_hi, dinv, W, b, act, last):
    n = g_lo.shape[0]
    hf = g_lo.shape[1]
    fout = W.shape[1]
    fo = fout // 2
    if last:
        out_specs = [pl.BlockSpec((SEG, fout), lambda i: (i, 0))]
        out_shape = [jax.ShapeDtypeStruct((n, fout), jnp.float32)]
    else:
        out_specs = [
            pl.BlockSpec((SEG, fo), lambda i: (i, 0)),
            pl.BlockSpec((SEG, fo), lambda i: (i, 0)),
        ]
        out_shape = [
            jax.ShapeDtypeStruct((n, fo), jnp.float32),
            jax.ShapeDtypeStruct((n, fo), jnp.float32),
        ]
    flat = [r for lh in s_parts for r in lh]
    return pl.pallas_call(
        functools.partial(_layer_body, act=act, last=last),
        grid=(NSEG,),
        in_specs=[_seg_spec(hf)] * 6 + [
            pl.BlockSpec((SEG, hf), lambda i: (i, 0)),
            pl.BlockSpec((SEG, hf), lambda i: (i, 0)),
            pl.BlockSpec((SEG, 1), lambda i: (i, 0)),
            pl.BlockSpec((2 * hf, fout), lambda i: (0, 0)),
            pl.BlockSpec((1, fout), lambda i: (0, 0)),
        ],
        out_specs=out_specs,
        out_shape=out_shape,
    )(*flat, g_lo, g_hi, dinv, W, b)


def _pool_body(h_ref, batch_ref, wl_ref, bl_ref, out_ref, acc_ref, cnt_ref):
    i = pl.program_id(0)

    @pl.when(i == 0)
    def _init():
        acc_ref[...] = jnp.zeros_like(acc_ref)
        cnt_ref[...] = jnp.zeros_like(cnt_ref)

    b = batch_ref[0, 0, :]
    gids = lax.broadcasted_iota(jnp.int32, (N_GRAPHS, POOL_BLK), 0)
    onehot = (gids == b[None, :]).astype(jnp.float32)
    acc_ref[...] += jnp.dot(onehot, h_ref[...], preferred_element_type=jnp.float32)
    cnt_ref[...] += jnp.sum(onehot, axis=1, keepdims=True)

    @pl.when(i == pl.num_programs(0) - 1)
    def _fin():
        pooled = acc_ref[...] / jnp.maximum(cnt_ref[...], 1.0)
        out_ref[...] = (
            jnp.dot(pooled, wl_ref[...], preferred_element_type=jnp.float32)
            + bl_ref[...]
        )


def _pool_classify(h, batch, Wl, bl):
    n, f = h.shape
    ncls = Wl.shape[1]
    batch3d = batch.astype(jnp.int32).reshape(n // POOL_BLK, 1, POOL_BLK)
    return pl.pallas_call(
        _pool_body,
        grid=(n // POOL_BLK,),
        in_specs=[
            pl.BlockSpec((POOL_BLK, f), lambda i: (i, 0)),
            pl.BlockSpec((1, 1, POOL_BLK), lambda i: (i, 0, 0)),
            pl.BlockSpec((f, ncls), lambda i: (0, 0)),
            pl.BlockSpec((1, ncls), lambda i: (0, 0)),
        ],
        out_specs=pl.BlockSpec((N_GRAPHS, ncls), lambda i: (0, 0)),
        out_shape=jax.ShapeDtypeStruct((N_GRAPHS, ncls), jnp.float32),
        scratch_shapes=[
            pltpu.VMEM((N_GRAPHS, f), jnp.float32),
            pltpu.VMEM((N_GRAPHS, 1), jnp.float32),
        ],
    )(h, batch3d, Wl, bl.reshape(1, ncls))


def kernel(x, edge_index, batch, W1, b1, W2, b2, W3, b3, W4, b4, Wl, bl):
    src = edge_index[0].astype(jnp.int32)
    dst = edge_index[1].astype(jnp.int32)
    npad = E_PAD - N_EDGES
    pad = jnp.arange(npad, dtype=jnp.int32) % 16
    src_p = jnp.concatenate([src, pad])
    dst_p = jnp.concatenate([dst, N_NODES + pad])
    key = (dst_p >= SEG).astype(jnp.int32) + (dst_p >= 2 * SEG).astype(jnp.int32)
    packed = src_p * 16384 + dst_p
    _, packed = lax.sort((key, packed), num_keys=1)
    src_p = packed >> 14
    dst_p = packed & 16383
    src2d = src_p.reshape(IDX_ROWS, 128)
    dst2d = dst_p.reshape(IDX_ROWS, 128)
    dump = SEG + (jnp.arange(E_PAD, dtype=jnp.int32) % (ACC_ROWS - SEG))
    dsts = []
    for k in range(NSEG):
        lo = k * SEG
        hi = min(lo + SEG, N_NODES)
        in_seg = (dst_p >= lo) & (dst_p < hi)
        dsts.append(jnp.where(in_seg, dst_p - lo, dump).reshape(IDX_ROWS, 128))

    n0 = jnp.sum((key == 0).astype(jnp.int32))
    n1 = jnp.sum((key == 1).astype(jnp.int32))
    n2 = E_PAD - n0 - n1
    starts = [jnp.int32(0), n0, n0 + n1]
    rks = [jnp.clip(st // 128, 0, IDX_ROWS - CAP_ROWS) for st in starts]
    src_w = [lax.dynamic_slice(src2d, (rk, 0), (CAP_ROWS, 128)) for rk in rks]
    dst_w = [lax.dynamic_slice(dsts[k], (rks[k], 0), (CAP_ROWS, 128))
             for k in range(NSEG)]
    fits = (n0 <= CAP_E - 128) & (n1 <= CAP_E - 128) & (n2 <= CAP_E - 128)

    deg0, deg1 = _deg_sc(dst2d)
    d0 = deg0[:N_NODES].reshape(N_NODES, 1)
    d1 = deg1[:N_NODES].reshape(N_NODES, 1)

    dinv, g0 = _dinv_tc(x, d0, d1)

    def _chain(agg1, agg):
        s_parts = agg1(g0)
        g_lo, g_hi = _layer1_tc(s_parts, g0, dinv, W1, b1.reshape(1, -1))
        layers = ((W2, b2, True), (W3, b3, True), (W4, b4, False))
        h = None
        for li, (W, b, act) in enumerate(layers):
            s_parts = agg(g_lo, g_hi)
            last = li == len(layers) - 1
            outs = _layer_tc(s_parts, g_lo, g_hi, dinv, W, b.reshape(1, -1),
                             act=act, last=last)
            if last:
                h = outs[0]
            else:
                g_lo, g_hi = outs
        return h

    def _fast():
        return _chain(
            lambda g: [_agg_sc_esplit(g, src_w[k], dst_w[k])
                       for k in range(NSEG)],
            lambda gl, gh: [_agg_sc(gl, gh, src_w[k], dst_w[k])
                            for k in range(NSEG)],
        )

    def _slow():
        return _chain(
            lambda g: [_agg_sc_esplit(g, src2d, dsts[k])
                       for k in range(NSEG)],
            lambda gl, gh: [_agg_sc(gl, gh, src2d, dsts[k])
                            for k in range(NSEG)],
        )

    h = lax.cond(fits, _fast, _slow)
    return _pool_classify(h, batch, Wl, bl)


# speedup vs baseline: 1.3316x; 1.2220x over previous
import functools

import jax
import jax.numpy as jnp
from jax import lax
from jax.experimental import pallas as pl
from jax.experimental.pallas import tpu as pltpu
from jax.experimental.pallas import tpu_sc as plsc

N_NODES = 10000
N_GRAPHS = 64
N_EDGES = 320000

E_PAD = 327680
IDX_ROWS = E_PAD // 128
CAP_ROWS = 1024
CAP_E = CAP_ROWS * 128
SEG = 3392
NSEG = 3
ACC_ROWS = 3456
DEG_ROWS = 10240
POOL_BLK = 2000

_MESH = plsc.VectorSubcoreMesh(core_axis_name="c", subcore_axis_name="s")


def _deg_sc(dst2d):
    nrt = IDX_ROWS // 32
    zr = DEG_ROWS // 16

    @functools.partial(
        pl.kernel,
        out_type=[
            jax.ShapeDtypeStruct((DEG_ROWS,), jnp.float32),
            jax.ShapeDtypeStruct((DEG_ROWS,), jnp.float32),
        ],
        mesh=_MESH,
        scratch_types=[
            pltpu.VMEM((nrt, 128), jnp.int32),
            pltpu.VMEM((128,), jnp.float32),
            pltpu.VMEM((zr,), jnp.float32),
            pltpu.VMEM_SHARED((DEG_ROWS,), jnp.float32),
        ],
    )
    def deg_kernel(dst_hbm, out0, out1, didx, ones, zbuf, acc):
        c = lax.axis_index("c")
        s = lax.axis_index("s")

        def fill(i, carry):
            zbuf[pl.ds(i * 16, 16)] = jnp.zeros((16,), jnp.float32)
            return carry

        lax.fori_loop(0, zr // 16, fill, 0)
        for j in range(8):
            ones[pl.ds(j * 16, 16)] = jnp.ones((16,), jnp.float32)

        pltpu.sync_copy(zbuf, acc.at[pl.ds(s * zr, zr)])
        wbase = (c * 16 + s) * nrt
        pltpu.sync_copy(dst_hbm.at[pl.ds(wbase, nrt)], didx)
        plsc.subcore_barrier()

        def body(j, carry):
            pltpu.sync_copy(ones, acc.at[didx.at[j]], add=True)
            return carry

        lax.fori_loop(0, nrt, body, 0)
        plsc.subcore_barrier()

        @pl.when(c == 0)
        def _w0():
            pltpu.sync_copy(acc.at[pl.ds(s * zr, zr)], out0.at[pl.ds(s * zr, zr)])

        @pl.when(c == 1)
        def _w1():
            pltpu.sync_copy(acc.at[pl.ds(s * zr, zr)], out1.at[pl.ds(s * zr, zr)])

    return deg_kernel(dst2d)


def _agg_body(tab_for_core, src_hbm, dst_hbm, outs,
              sidx, didx, rows, zbuf, acc, gsems, *, nrt, edge_split, nbuf):
    c = lax.axis_index("c")
    s = lax.axis_index("s")
    f = zbuf.shape[1]
    zr = ACC_ROWS // 16
    zh = zr // 3

    def fill(i, carry):
        for j in range(f // 16):
            zbuf[i, pl.ds(j * 16, 16)] = jnp.zeros((16,), jnp.float32)
        return carry

    lax.fori_loop(0, zh, fill, 0)
    base = s * zr
    for t in range(3):
        pltpu.sync_copy(zbuf, acc.at[pl.ds(base + t * zh, zh)])

    if edge_split:
        rbase = (c * 16 + s) * nrt
    else:
        rbase = s * nrt
    pltpu.sync_copy(src_hbm.at[pl.ds(rbase, nrt)], sidx)
    pltpu.sync_copy(dst_hbm.at[pl.ds(rbase, nrt)], didx)
    plsc.subcore_barrier()

    def edge_loop(tab):
        for b in range(nbuf - 1):
            pltpu.async_copy(tab.at[sidx.at[b]], rows.at[b], gsems[b])

        def body(i, carry):
            for b in range(nbuf):
                j = nbuf * i + b
                bw = (b + nbuf - 1) % nbuf
                pltpu.make_async_copy(
                    tab.at[sidx.at[j]], rows.at[b], gsems[b]).wait()
                pltpu.sync_copy(rows.at[b], acc.at[didx.at[j]], add=True)

                @pl.when(j + nbuf - 1 < nrt)
                def _prefetch():
                    pltpu.async_copy(tab.at[sidx.at[j + nbuf - 1]],
                                     rows.at[bw], gsems[bw])
            return carry

        lax.fori_loop(0, nrt // nbuf, body, 0)

    if edge_split:
        edge_loop(tab_for_core[0])
    else:
        @pl.when(c == 0)
        def _e0():
            edge_loop(tab_for_core[0])

        @pl.when(c == 1)
        def _e1():
            edge_loop(tab_for_core[1])

    plsc.subcore_barrier()

    @pl.when(c == 0)
    def _w0():
        pltpu.sync_copy(acc.at[pl.ds(base, zr)], outs[0].at[pl.ds(base, zr)])

    @pl.when(c == 1)
    def _w1():
        pltpu.sync_copy(acc.at[pl.ds(base, zr)], outs[1].at[pl.ds(base, zr)])


def _agg_scratch(f, nrt, nbuf):
    return [
        pltpu.VMEM((nrt, 128), jnp.int32),
        pltpu.VMEM((nrt, 128), jnp.int32),
        pltpu.VMEM((nbuf, 128, f), jnp.float32),
        pltpu.VMEM((ACC_ROWS // 48, f), jnp.float32),
        pltpu.VMEM_SHARED((ACC_ROWS, f), jnp.float32),
    ] + [pltpu.SemaphoreType.DMA] * nbuf


def _agg_sc(g_lo, g_hi, src2d, dsts2d, nbuf):
    f = g_lo.shape[1]
    nrt = src2d.shape[0] // 16

    @functools.partial(
        pl.kernel,
        out_type=[jax.ShapeDtypeStruct((ACC_ROWS, f), jnp.float32)] * 2,
        mesh=_MESH,
        scratch_types=_agg_scratch(f, nrt, nbuf),
    )
    def agg_kernel(glo_hbm, ghi_hbm, src_hbm, dst_hbm, out_lo, out_hi,
                   sidx, didx, rows, zbuf, acc, *sems):
        _agg_body((glo_hbm, ghi_hbm), src_hbm, dst_hbm, (out_lo, out_hi),
                  sidx, didx, rows, zbuf, acc, sems,
                  nrt=nrt, edge_split=False, nbuf=nbuf)

    return agg_kernel(g_lo, g_hi, src2d, dsts2d)


def _agg_sc_esplit(g, src2d, dsts2d, nbuf):
    f = g.shape[1]
    nrt = src2d.shape[0] // 32

    @functools.partial(
        pl.kernel,
        out_type=[jax.ShapeDtypeStruct((ACC_ROWS, f), jnp.float32)] * 2,
        mesh=_MESH,
        scratch_types=_agg_scratch(f, nrt, nbuf),
    )
    def agg_kernel(g_hbm, src_hbm, dst_hbm, out_a, out_b,
                   sidx, didx, rows, zbuf, acc, *sems):
        _agg_body((g_hbm,), src_hbm, dst_hbm, (out_a, out_b),
                  sidx, didx, rows, zbuf, acc, sems,
                  nrt=nrt, edge_split=True, nbuf=nbuf)

    return agg_kernel(g, src2d, dsts2d)


def _dinv_body(x_ref, d0_ref, d1_ref, dinv_ref, g_ref):
    dinv = lax.rsqrt(1.0 + d0_ref[...] + d1_ref[...])
    dinv_ref[...] = dinv
    g_ref[...] = dinv * x_ref[...]


def _dinv_tc(x, d0, d1):
    n, fin = x.shape
    return pl.pallas_call(
        _dinv_body,
        grid=(NSEG,),
        in_specs=[
            pl.BlockSpec((SEG, fin), lambda i: (i, 0)),
            pl.BlockSpec((SEG, 1), lambda i: (i, 0)),
            pl.BlockSpec((SEG, 1), lambda i: (i, 0)),
        ],
        out_specs=[
            pl.BlockSpec((SEG, 1), lambda i: (i, 0)),
            pl.BlockSpec((SEG, fin), lambda i: (i, 0)),
        ],
        out_shape=[
            jax.ShapeDtypeStruct((n, 1), jnp.float32),
            jax.ShapeDtypeStruct((n, fin), jnp.float32),
        ],
    )(x, d0, d1)


def _seg_spec(f):
    return pl.BlockSpec((SEG, f), lambda i: (0, 0))


def _pick3(i, r0, r1, r2):
    return jnp.where(i == 0, r0[...], jnp.where(i == 1, r1[...], r2[...]))


def _layer1_body(s0a_ref, s0b_ref, s1a_ref, s1b_ref, s2a_ref, s2b_ref,
                 g_ref, dinv_ref, w_ref, b_ref, glo_ref, ghi_ref):
    i = pl.program_id(0)
    sa = _pick3(i, s0a_ref, s1a_ref, s2a_ref)
    sb = _pick3(i, s0b_ref, s1b_ref, s2b_ref)
    dinv = dinv_ref[...]
    t = dinv * (sa + sb + g_ref[...])
    h = jnp.dot(t, w_ref[...], preferred_element_type=jnp.float32)
    h = jnp.maximum(h + b_ref[...], 0.0)
    g = dinv * h
    fo = h.shape[1] // 2
    glo_ref[...] = g[:, :fo]
    ghi_ref[...] = g[:, fo:]


def _layer1_tc(s_parts, g, dinv, W, b):
    n, fin = g.shape
    fout = W.shape[1]
    fo = fout // 2
    flat = [r for ab in s_parts for r in ab]
    return pl.pallas_call(
        _layer1_body,
        grid=(NSEG,),
        in_specs=[_seg_spec(fin)] * 6 + [
            pl.BlockSpec((SEG, fin), lambda i: (i, 0)),
            pl.BlockSpec((SEG, 1), lambda i: (i, 0)),
            pl.BlockSpec((fin, fout), lambda i: (0, 0)),
            pl.BlockSpec((1, fout), lambda i: (0, 0)),
        ],
        out_specs=[
            pl.BlockSpec((SEG, fo), lambda i: (i, 0)),
            pl.BlockSpec((SEG, fo), lambda i: (i, 0)),
        ],
        out_shape=[
            jax.ShapeDtypeStruct((n, fo), jnp.float32),
            jax.ShapeDtypeStruct((n, fo), jnp.float32),
        ],
    )(*flat, g, dinv, W, b)


def _layer_body(s0l_ref, s0h_ref, s1l_ref, s1h_ref, s2l_ref, s2h_ref,
                glo_ref, ghi_ref, dinv_ref, w_ref, b_ref, *out_refs,
                act, last):
    i = pl.program_id(0)
    s_lo = _pick3(i, s0l_ref, s1l_ref, s2l_ref)
    s_hi = _pick3(i, s0h_ref, s1h_ref, s2h_ref)
    dinv = dinv_ref[...]
    hf = s_lo.shape[1]
    t_lo = dinv * (s_lo + glo_ref[...])
    t_hi = dinv * (s_hi + ghi_ref[...])
    h = jnp.dot(t_lo, w_ref[:hf, :], preferred_element_type=jnp.float32)
    h += jnp.dot(t_hi, w_ref[hf:, :], preferred_element_type=jnp.float32)
    h = h + b_ref[...]
    if act:
        h = jnp.maximum(h, 0.0)
    if last:
        out_refs[0][...] = h
    else:
        fo = h.shape[1] // 2
        g = dinv * h
        out_refs[0][...] = g[:, :fo]
        out_refs[1][...] = g[:, fo:]


def _layer_tc(s_parts, g_lo, g_hi, dinv, W, b, act, last):
    n = g_lo.shape[0]
    hf = g_lo.shape[1]
    fout = W.shape[1]
    fo = fout // 2
    if last:
        out_specs = [pl.BlockSpec((SEG, fout), lambda i: (i, 0))]
        out_shape = [jax.ShapeDtypeStruct((n, fout), jnp.float32)]
    else:
        out_specs = [
            pl.BlockSpec((SEG, fo), lambda i: (i, 0)),
            pl.BlockSpec((SEG, fo), lambda i: (i, 0)),
        ]
        out_shape = [
            jax.ShapeDtypeStruct((n, fo), jnp.float32),
            jax.ShapeDtypeStruct((n, fo), jnp.float32),
        ]
    flat = [r for lh in s_parts for r in lh]
    return pl.pallas_call(
        functools.partial(_layer_body, act=act, last=last),
        grid=(NSEG,),
        in_specs=[_seg_spec(hf)] * 6 + [
            pl.BlockSpec((SEG, hf), lambda i: (i, 0)),
            pl.BlockSpec((SEG, hf), lambda i: (i, 0)),
            pl.BlockSpec((SEG, 1), lambda i: (i, 0)),
            pl.BlockSpec((2 * hf, fout), lambda i: (0, 0)),
            pl.BlockSpec((1, fout), lambda i: (0, 0)),
        ],
        out_specs=out_specs,
        out_shape=out_shape,
    )(*flat, g_lo, g_hi, dinv, W, b)


def _pool_body(h_ref, batch_ref, wl_ref, bl_ref, out_ref, acc_ref, cnt_ref):
    i = pl.program_id(0)

    @pl.when(i == 0)
    def _init():
        acc_ref[...] = jnp.zeros_like(acc_ref)
        cnt_ref[...] = jnp.zeros_like(cnt_ref)

    b = batch_ref[0, 0, :]
    gids = lax.broadcasted_iota(jnp.int32, (N_GRAPHS, POOL_BLK), 0)
    onehot = (gids == b[None, :]).astype(jnp.float32)
    acc_ref[...] += jnp.dot(onehot, h_ref[...], preferred_element_type=jnp.float32)
    cnt_ref[...] += jnp.sum(onehot, axis=1, keepdims=True)

    @pl.when(i == pl.num_programs(0) - 1)
    def _fin():
        pooled = acc_ref[...] / jnp.maximum(cnt_ref[...], 1.0)
        out_ref[...] = (
            jnp.dot(pooled, wl_ref[...], preferred_element_type=jnp.float32)
            + bl_ref[...]
        )


def _pool_classify(h, batch, Wl, bl):
    n, f = h.shape
    ncls = Wl.shape[1]
    batch3d = batch.astype(jnp.int32).reshape(n // POOL_BLK, 1, POOL_BLK)
    return pl.pallas_call(
        _pool_body,
        grid=(n // POOL_BLK,),
        in_specs=[
            pl.BlockSpec((POOL_BLK, f), lambda i: (i, 0)),
            pl.BlockSpec((1, 1, POOL_BLK), lambda i: (i, 0, 0)),
            pl.BlockSpec((f, ncls), lambda i: (0, 0)),
            pl.BlockSpec((1, ncls), lambda i: (0, 0)),
        ],
        out_specs=pl.BlockSpec((N_GRAPHS, ncls), lambda i: (0, 0)),
        out_shape=jax.ShapeDtypeStruct((N_GRAPHS, ncls), jnp.float32),
        scratch_shapes=[
            pltpu.VMEM((N_GRAPHS, f), jnp.float32),
            pltpu.VMEM((N_GRAPHS, 1), jnp.float32),
        ],
    )(h, batch3d, Wl, bl.reshape(1, ncls))


def kernel(x, edge_index, batch, W1, b1, W2, b2, W3, b3, W4, b4, Wl, bl):
    src = edge_index[0].astype(jnp.int32)
    dst = edge_index[1].astype(jnp.int32)
    npad = E_PAD - N_EDGES
    pad = jnp.arange(npad, dtype=jnp.int32) % 16
    src_p = jnp.concatenate([src, pad])
    dst_p = jnp.concatenate([dst, N_NODES + pad])
    key = (dst_p >= SEG).astype(jnp.int32) + (dst_p >= 2 * SEG).astype(jnp.int32)
    packed = src_p * 16384 + dst_p
    _, packed = lax.sort((key, packed), num_keys=1)
    src_p = packed >> 14
    dst_p = packed & 16383
    src2d = src_p.reshape(IDX_ROWS, 128)
    dst2d = dst_p.reshape(IDX_ROWS, 128)
    dump = SEG + (jnp.arange(E_PAD, dtype=jnp.int32) % (ACC_ROWS - SEG))
    dsts = []
    for k in range(NSEG):
        lo = k * SEG
        hi = min(lo + SEG, N_NODES)
        in_seg = (dst_p >= lo) & (dst_p < hi)
        dsts.append(jnp.where(in_seg, dst_p - lo, dump).reshape(IDX_ROWS, 128))

    n0 = jnp.sum((key == 0).astype(jnp.int32))
    n1 = jnp.sum((key == 1).astype(jnp.int32))
    n2 = E_PAD - n0 - n1
    starts = [jnp.int32(0), n0, n0 + n1]
    rks = [jnp.clip(st // 128, 0, IDX_ROWS - CAP_ROWS) for st in starts]
    src_w = [lax.dynamic_slice(src2d, (rk, 0), (CAP_ROWS, 128)) for rk in rks]
    dst_w = [lax.dynamic_slice(dsts[k], (rks[k], 0), (CAP_ROWS, 128))
             for k in range(NSEG)]
    fits = (n0 <= CAP_E - 128) & (n1 <= CAP_E - 128) & (n2 <= CAP_E - 128)

    deg0, deg1 = _deg_sc(dst2d)
    d0 = deg0[:N_NODES].reshape(N_NODES, 1)
    d1 = deg1[:N_NODES].reshape(N_NODES, 1)

    dinv, g0 = _dinv_tc(x, d0, d1)

    def _chain(agg1, agg):
        s_parts = agg1(g0)
        g_lo, g_hi = _layer1_tc(s_parts, g0, dinv, W1, b1.reshape(1, -1))
        layers = ((W2, b2, True), (W3, b3, True), (W4, b4, False))
        h = None
        for li, (W, b, act) in enumerate(layers):
            s_parts = agg(g_lo, g_hi)
            last = li == len(layers) - 1
            outs = _layer_tc(s_parts, g_lo, g_hi, dinv, W, b.reshape(1, -1),
                             act=act, last=last)
            if last:
                h = outs[0]
            else:
                g_lo, g_hi = outs
        return h

    def _fast():
        return _chain(
            lambda g: [_agg_sc_esplit(g, src_w[k], dst_w[k], 4)
                       for k in range(NSEG)],
            lambda gl, gh: [_agg_sc(gl, gh, src_w[k], dst_w[k], 4)
                            for k in range(NSEG)],
        )

    def _slow():
        return _chain(
            lambda g: [_agg_sc_esplit(g, src2d, dsts[k], 2)
                       for k in range(NSEG)],
            lambda gl, gh: [_agg_sc(gl, gh, src2d, dsts[k], 2)
                            for k in range(NSEG)],
        )

    h = lax.cond(fits, _fast, _slow)
    return _pool_classify(h, batch, Wl, bl)
